# baseline (speedup 1.0000x reference)
import jax
import jax.numpy as jnp
from jax import lax
from jax.experimental import pallas as pl
from jax.experimental.pallas import tpu as pltpu
from jax.experimental.pallas import tpu_sc as plsc

_BATCH = 4096
_HIST = 200
_D = 64
_B = _BATCH * _HIST
_NC = 2
_NS = 16
_NW = _NC * _NS
_PER_W = _B // _NW
_IDXW = 128
_SUB = 4
_C = _SUB * _IDXW
_SEGS = _BATCH // _C
_NCHUNK = _PER_W // _C
_T = _NCHUNK // 2


def _emb_body(ids_hbm, emb_hbm, mask_hbm, out_hbm, mout_hbm,
              idx_v0, idx_v1, rows_v0, rows_v1, mrows_v0, mrows_v1,
              sem_i0, sem_i1, sem_g0, sem_g1, sem_w0, sem_w1):
    wid = lax.axis_index("s") * _NC + lax.axis_index("c")
    chunk_base = wid * _NCHUNK
    mask1d = mask_hbm.at[0]
    idx_v = (idx_v0, idx_v1)
    rows_v = (rows_v0, rows_v1)
    mrows_v = (mrows_v0, mrows_v1)
    sem_i = (sem_i0, sem_i1)
    sem_g = (sem_g0, sem_g1)
    sem_w = (sem_w0, sem_w1)

    def idx_desc(g, b):
        f = chunk_base + g
        h = f // _SEGS
        b0 = (f % _SEGS) * _C
        return pltpu.make_async_copy(
            ids_hbm.at[pl.ds(h, 1), pl.ds(b0, _C)], idx_v[b], sem_i[b])

    def fire_gathers(b):
        copies = []
        for j in range(_SUB):
            iv = idx_v[b].at[0, pl.ds(j * _IDXW, _IDXW)]
            copies.append(pltpu.async_copy(
                emb_hbm.at[iv],
                rows_v[b].at[0, pl.ds(j * _IDXW, _IDXW), :], sem_g[b]))
            copies.append(pltpu.async_copy(
                mask1d.at[iv],
                mrows_v[b].at[0, pl.ds(j * _IDXW, _IDXW)], sem_g[b]))
        return copies

    def wb_descs(g, b):
        f = chunk_base + g
        h = f // _SEGS
        b0 = (f % _SEGS) * _C
        off = f * _C
        return (
            pltpu.make_async_copy(rows_v[b],
                                  out_hbm.at[pl.ds(h, 1), pl.ds(b0, _C), :],
                                  sem_w[b]),
            pltpu.make_async_copy(mrows_v[b],
                                  mout_hbm.at[pl.ds(h, 1), pl.ds(b0, _C)],
                                  sem_w[b]),
        )

    def fire_wb(g, b):
        for dsc in wb_descs(g, b):
            dsc.start()

    def wait_wb(g, b):
        for dsc in wb_descs(g, b):
            dsc.wait()

    idx_desc(0, 0).start()
    idx_desc(1, 1).start()
    idx_desc(0, 0).wait()
    d0 = fire_gathers(0)
    idx_desc(1, 1).wait()
    d1 = fire_gathers(1)
    for dsc in d0:
        dsc.wait()
    fire_wb(0, 0)
    idx_desc(2, 0).start()
    for dsc in d1:
        dsc.wait()
    fire_wb(1, 1)
    idx_desc(3, 1).start()

    def step(t, carry):
        ga = 2 * t
        gb = 2 * t + 1
        na = jnp.minimum(ga + 2, _NCHUNK - 1)
        nb = jnp.minimum(gb + 2, _NCHUNK - 1)
        wait_wb(ga - 2, 0)
        idx_desc(ga, 0).wait()
        da = fire_gathers(0)
        wait_wb(gb - 2, 1)
        idx_desc(gb, 1).wait()
        db = fire_gathers(1)
        for dsc in da:
            dsc.wait()
        fire_wb(ga, 0)
        idx_desc(na, 0).start()
        for dsc in db:
            dsc.wait()
        fire_wb(gb, 1)
        idx_desc(nb, 1).start()
        return carry

    lax.fori_loop(1, _T, step, 0)

    wait_wb(2 * _T - 2, 0)
    wait_wb(2 * _T - 1, 1)
    idx_desc(_NCHUNK - 1, 0).wait()
    idx_desc(_NCHUNK - 1, 1).wait()


def kernel(input_ids, emb_table, mask_table):
    ids_t = input_ids.T
    mask_t = mask_table.T
    mesh = plsc.VectorSubcoreMesh(
        core_axis_name="c", subcore_axis_name="s",
        num_cores=_NC, num_subcores=_NS)
    emb_flat, mask2d = pl.kernel(
        _emb_body,
        out_type=(
            jax.ShapeDtypeStruct((_HIST, _BATCH, _D), jnp.float32),
            jax.ShapeDtypeStruct((_HIST, _BATCH), jnp.float32),
        ),
        mesh=mesh,
        compiler_params=pltpu.CompilerParams(use_tc_tiling_on_sc=False),
        scratch_types=[
            pltpu.VMEM((1, _C), jnp.int32),
            pltpu.VMEM((1, _C), jnp.int32),
            pltpu.VMEM((1, _C, _D), jnp.float32),
            pltpu.VMEM((1, _C, _D), jnp.float32),
            pltpu.VMEM((1, _C), jnp.float32),
            pltpu.VMEM((1, _C), jnp.float32),
            pltpu.SemaphoreType.DMA,
            pltpu.SemaphoreType.DMA,
            pltpu.SemaphoreType.DMA,
            pltpu.SemaphoreType.DMA,
            pltpu.SemaphoreType.DMA,
            pltpu.SemaphoreType.DMA,
        ],
    )(ids_t, emb_table, mask_t)
    embedding = emb_flat.transpose(1, 0, 2)
    mask = mask2d.T.astype(jnp.bool_)
    return (embedding, mask)


# speedup vs baseline: 1.0394x; 1.0394x over previous
import jax
import jax.numpy as jnp
from jax import lax
from jax.experimental import pallas as pl
from jax.experimental.pallas import tpu as pltpu
from jax.experimental.pallas import tpu_sc as plsc

_BATCH = 4096
_HIST = 200
_D = 64
_B = _BATCH * _HIST
_NC = 2
_NS = 16
_NW = _NC * _NS
_PER_W = _B // _NW
_IDXW = 128
_SUB = 4
_C = _SUB * _IDXW
_SEGS = _BATCH // _C
_NCHUNK = _PER_W // _C
_T = _NCHUNK // 2


def _emb_body(ids_hbm, emb_hbm, mask_hbm, out_hbm, mout_hbm,
              idx_v0, idx_v1, rows_v0, rows_v1, mrows_v0, mrows_v1,
              sem_i0, sem_i1, sem_g0, sem_g1, sem_w0, sem_w1):
    wid = lax.axis_index("s") * _NC + lax.axis_index("c")
    chunk_base = wid * _NCHUNK
    mask1d = mask_hbm.at[0]
    idx_v = (idx_v0, idx_v1)
    rows_v = (rows_v0, rows_v1)
    mrows_v = (mrows_v0, mrows_v1)
    sem_i = (sem_i0, sem_i1)
    sem_g = (sem_g0, sem_g1)
    sem_w = (sem_w0, sem_w1)

    def idx_desc(g, b):
        f = chunk_base + g
        h = f // _SEGS
        b0 = (f % _SEGS) * _C
        return pltpu.make_async_copy(
            ids_hbm.at[pl.ds(h, 1), pl.ds(b0, _C)], idx_v[b], sem_i[b])

    def fire_gathers(b):
        copies = []
        for j in range(_SUB):
            iv = idx_v[b].at[0, pl.ds(j * _IDXW, _IDXW)]
            copies.append(pltpu.async_copy(
                emb_hbm.at[iv],
                rows_v[b].at[0, pl.ds(j * _IDXW, _IDXW), :], sem_g[b]))
            copies.append(pltpu.async_copy(
                mask1d.at[iv],
                mrows_v[b].at[0, pl.ds(j * _IDXW, _IDXW)], sem_g[b]))
        return copies

    def wb_descs(g, b):
        f = chunk_base + g
        h = f // _SEGS
        b0 = (f % _SEGS) * _C
        off = f * _C
        return (
            pltpu.make_async_copy(rows_v[b],
                                  out_hbm.at[pl.ds(h, 1), pl.ds(b0, _C),
                                             pl.ds(0, _D)],
                                  sem_w[b]),
            pltpu.make_async_copy(mrows_v[b],
                                  mout_hbm.at[pl.ds(h, 1), pl.ds(b0, _C)],
                                  sem_w[b]),
        )

    def fire_wb(g, b):
        for dsc in wb_descs(g, b):
            dsc.start()

    def wait_wb(g, b):
        for dsc in wb_descs(g, b):
            dsc.wait()

    idx_desc(0, 0).start()
    idx_desc(1, 1).start()
    idx_desc(0, 0).wait()
    d0 = fire_gathers(0)
    idx_desc(1, 1).wait()
    d1 = fire_gathers(1)
    for dsc in d0:
        dsc.wait()
    fire_wb(0, 0)
    idx_desc(2, 0).start()
    for dsc in d1:
        dsc.wait()
    fire_wb(1, 1)
    idx_desc(3, 1).start()

    def step(t, carry):
        ga = 2 * t
        gb = 2 * t + 1
        na = jnp.minimum(ga + 2, _NCHUNK - 1)
        nb = jnp.minimum(gb + 2, _NCHUNK - 1)
        wait_wb(ga - 2, 0)
        idx_desc(ga, 0).wait()
        da = fire_gathers(0)
        wait_wb(gb - 2, 1)
        idx_desc(gb, 1).wait()
        db = fire_gathers(1)
        for dsc in da:
            dsc.wait()
        fire_wb(ga, 0)
        idx_desc(na, 0).start()
        for dsc in db:
            dsc.wait()
        fire_wb(gb, 1)
        idx_desc(nb, 1).start()
        return carry

    lax.fori_loop(1, _T, step, 0)

    wait_wb(2 * _T - 2, 0)
    wait_wb(2 * _T - 1, 1)
    idx_desc(_NCHUNK - 1, 0).wait()
    idx_desc(_NCHUNK - 1, 1).wait()


def kernel(input_ids, emb_table, mask_table):
    ids_t = input_ids.T
    mask_t = mask_table.T
    mesh = plsc.VectorSubcoreMesh(
        core_axis_name="c", subcore_axis_name="s",
        num_cores=_NC, num_subcores=_NS)
    emb_flat, mask2d = pl.kernel(
        _emb_body,
        out_type=(
            jax.ShapeDtypeStruct((_HIST, _BATCH, 2 * _D), jnp.float32),
            jax.ShapeDtypeStruct((_HIST, _BATCH), jnp.float32),
        ),
        mesh=mesh,
        compiler_params=pltpu.CompilerParams(use_tc_tiling_on_sc=False),
        scratch_types=[
            pltpu.VMEM((1, _C), jnp.int32),
            pltpu.VMEM((1, _C), jnp.int32),
            pltpu.VMEM((1, _C, _D), jnp.float32),
            pltpu.VMEM((1, _C, _D), jnp.float32),
            pltpu.VMEM((1, _C), jnp.float32),
            pltpu.VMEM((1, _C), jnp.float32),
            pltpu.SemaphoreType.DMA,
            pltpu.SemaphoreType.DMA,
            pltpu.SemaphoreType.DMA,
            pltpu.SemaphoreType.DMA,
            pltpu.SemaphoreType.DMA,
            pltpu.SemaphoreType.DMA,
        ],
    )(ids_t, emb_table, mask_t)
    embedding = emb_flat[:, :, :_D].transpose(1, 0, 2)
    mask = mask2d.T.astype(jnp.bool_)
    return (embedding, mask)
